# baseline (speedup 1.0000x reference)
import jax
import jax.numpy as jnp
from jax.experimental import pallas as pl
from jax.experimental.pallas import tpu as pltpu

_FEAT = 32
_NB = 2
_NR = 5


def _gae_body(u_ref, i_ref, b_ref, ct_ref, out_ref):
    u = u_ref[...]
    it = i_ref[...]
    a0 = jnp.dot(u, b_ref[0], preferred_element_type=jnp.float32)
    a1 = jnp.dot(u, b_ref[1], preferred_element_type=jnp.float32)
    p0t = jax.lax.dot_general(it, a0, (((1,), (1,)), ((), ())),
                              preferred_element_type=jnp.float32)
    p1t = jax.lax.dot_general(it, a1, (((1,), (1,)), ((), ())),
                              preferred_element_type=jnp.float32)
    c0 = ct_ref[0:1, :]
    c1 = ct_ref[1:2, :]
    for uu in range(out_ref.shape[0]):
        out_ref[uu] = p0t[:, uu:uu + 1] * c0 + p1t[:, uu:uu + 1] * c1


def kernel(u_features, i_features, basis_matrix, coefs):
    num_u, feat = u_features.shape
    num_i = i_features.shape[0]
    basis3 = basis_matrix.reshape(_NB, feat, feat)
    coefs_t = coefs.T
    tu, ti = 32, 512
    grid = (num_u // tu, num_i // ti)
    out = pl.pallas_call(
        _gae_body,
        grid=grid,
        in_specs=[
            pl.BlockSpec((tu, feat), lambda gu, gi: (gu, 0)),
            pl.BlockSpec((ti, feat), lambda gu, gi: (gi, 0)),
            pl.BlockSpec((_NB, feat, feat), lambda gu, gi: (0, 0, 0)),
            pl.BlockSpec((_NB, _NR), lambda gu, gi: (0, 0)),
        ],
        out_specs=pl.BlockSpec((tu, ti, _NR), lambda gu, gi: (gu, gi, 0)),
        out_shape=jax.ShapeDtypeStruct((num_u, num_i, _NR), jnp.float32),
    )(u_features, i_features, basis3, coefs_t)
    return out.reshape(num_u * num_i, _NR)


# speedup vs baseline: 6.1459x; 6.1459x over previous
import jax
import jax.numpy as jnp
from jax.experimental import pallas as pl
from jax.experimental.pallas import tpu as pltpu

_NB = 2
_NR = 5


def _gae_body(u_ref, i_ref, b_ref, c_ref, out_ref):
    u = u_ref[...]
    it = i_ref[...]
    a0 = jnp.dot(u, b_ref[0], preferred_element_type=jnp.float32)
    a1 = jnp.dot(u, b_ref[1], preferred_element_type=jnp.float32)
    c0 = c_ref[:, 0:1]
    c1 = c_ref[:, 1:2]
    ni = it.shape[0]
    for uu in range(u.shape[0]):
        g = c0 * a0[uu:uu + 1, :] + c1 * a1[uu:uu + 1, :]
        out_ref[:, uu * ni:(uu + 1) * ni] = jax.lax.dot_general(
            g, it, (((1,), (1,)), ((), ())),
            preferred_element_type=jnp.float32)


def kernel(u_features, i_features, basis_matrix, coefs):
    num_u, feat = u_features.shape
    num_i = i_features.shape[0]
    basis3 = basis_matrix.reshape(_NB, feat, feat)
    tu = 64
    grid = (num_u // tu,)
    out_t = pl.pallas_call(
        _gae_body,
        grid=grid,
        in_specs=[
            pl.BlockSpec((tu, feat), lambda g: (g, 0)),
            pl.BlockSpec((num_i, feat), lambda g: (0, 0)),
            pl.BlockSpec((_NB, feat, feat), lambda g: (0, 0, 0)),
            pl.BlockSpec((_NR, _NB), lambda g: (0, 0)),
        ],
        out_specs=pl.BlockSpec((_NR, tu * num_i), lambda g: (0, g)),
        out_shape=jax.ShapeDtypeStruct((_NR, num_u * num_i), jnp.float32),
    )(u_features, i_features, basis3, coefs)
    return out_t.T


# speedup vs baseline: 20.1081x; 3.2718x over previous
import numpy as np

import jax
import jax.numpy as jnp
from jax.experimental import pallas as pl
from jax.experimental.pallas import tpu as pltpu

_NB = 2
_NR = 5


def _gae_body(u_ref, i_ref, b_ref, s_ref, out_ref):
    u = u_ref[...]
    it = i_ref[...]
    a0 = jnp.dot(u, b_ref[0], preferred_element_type=jnp.float32)
    a1 = jnp.dot(u, b_ref[1], preferred_element_type=jnp.float32)
    acat = jnp.concatenate([a0, a1], axis=0)
    g8 = jnp.dot(s_ref[...], acat, preferred_element_type=jnp.float32)
    big = jax.lax.dot_general(g8, it, (((1,), (1,)), ((), ())),
                              preferred_element_type=jnp.float32)
    ni = it.shape[0]
    tu = u.shape[0]
    for uu in range(tu):
        out_ref[:, uu * ni:(uu + 1) * ni] = big[8 * uu:8 * uu + _NR, :]


def kernel(u_features, i_features, basis_matrix, coefs):
    num_u, feat = u_features.shape
    num_i = i_features.shape[0]
    basis3 = basis_matrix.reshape(_NB, feat, feat)
    tu = 64
    rows = (8 * np.arange(tu)[:, None] + np.arange(_NR)[None, :]).ravel()
    cols = np.repeat(np.arange(tu), _NR)
    sel = jnp.zeros((tu * 8, 2 * tu), jnp.float32)
    sel = sel.at[rows, cols].set(jnp.tile(coefs[:, 0], tu))
    sel = sel.at[rows, tu + cols].set(jnp.tile(coefs[:, 1], tu))
    grid = (num_u // tu,)
    out_t = pl.pallas_call(
        _gae_body,
        grid=grid,
        in_specs=[
            pl.BlockSpec((tu, feat), lambda g: (g, 0)),
            pl.BlockSpec((num_i, feat), lambda g: (0, 0)),
            pl.BlockSpec((_NB, feat, feat), lambda g: (0, 0, 0)),
            pl.BlockSpec((tu * 8, 2 * tu), lambda g: (0, 0)),
        ],
        out_specs=pl.BlockSpec((_NR, tu * num_i), lambda g: (0, g)),
        out_shape=jax.ShapeDtypeStruct((_NR, num_u * num_i), jnp.float32),
    )(u_features, i_features, basis3, sel)
    return out_t.T
